# Initial kernel scaffold; baseline (speedup 1.0000x reference)
import jax
import jax.numpy as jnp
from jax.experimental import pallas as pl


def kernel(x, target):
    raise NotImplementedError("write your pallas kernel here")


# speedup vs baseline: 17.8370x; 17.8370x over previous
import jax
import jax.numpy as jnp
from jax.experimental import pallas as pl
from jax.experimental.pallas import tpu as pltpu

_NBINS = 10
_ROWS = 16384
_COLS = 1000
_BR = 128
_NACC = 2 * (_NBINS - 1) + 1
_NCH = (_COLS + 127) // 128


def _ghme_body(x_ref, t_ref, out_ref, acc_ref):
    i = pl.program_id(0)
    nsteps = pl.num_programs(0)

    @pl.when(i == 0)
    def _init():
        acc_ref[...] = jnp.zeros_like(acc_ref)

    zero = jnp.zeros((8, 128), jnp.float32)

    tail = _COLS - (_NCH - 1) * 128
    pad = jnp.zeros((8, 128 - tail), jnp.float32)

    def row_group(rg, carry):
        accs = list(carry)
        r0 = rg * 8
        for c in range(_NCH):
            c0 = c * 128
            w = 128 if c < _NCH - 1 else tail
            x = x_ref[pl.ds(r0, 8), c0:c0 + w]
            t = t_ref[pl.ds(r0, 8), c0:c0 + w]
            g = jnp.abs(x - t)
            idx = jnp.floor(g * (_NBINS - 0.0001))
            lx = jnp.log(x)
            l1x = jnp.log(1.0 - x)
            nloss = t * (lx - l1x) + l1x
            if w < 128:
                idx = jnp.concatenate([idx, pad], axis=1)
                nloss = jnp.concatenate([nloss, pad], axis=1)
            accs[_NACC - 1] = accs[_NACC - 1] + nloss
            for j in range(_NBINS - 1):
                m01 = jnp.where(idx == jnp.float32(j + 1), 1.0, 0.0)
                accs[j] = accs[j] + m01 * nloss
                accs[_NBINS - 1 + j] = accs[_NBINS - 1 + j] + m01
        return tuple(accs)

    accs = jax.lax.fori_loop(0, _BR // 8, row_group, (zero,) * _NACC,
                             unroll=False)
    for k in range(_NACC):
        acc_ref[k] += accs[k]

    @pl.when(i == nsteps - 1)
    def _fini():
        n_total = jnp.float32(_ROWS * _COLS)
        sums = [jnp.sum(acc_ref[j]) for j in range(_NBINS - 1)]
        cnts = [jnp.sum(acc_ref[_NBINS - 1 + j]) for j in range(_NBINS - 1)]
        tot_s = jnp.sum(acc_ref[_NACC - 1])
        c_rest = cnts[0]
        s_rest = sums[0]
        for j in range(1, _NBINS - 1):
            c_rest += cnts[j]
            s_rest += sums[j]
        cnts = [n_total - c_rest] + cnts
        sums = [tot_s - s_rest] + sums
        nonempty = (cnts[0] > 0.0).astype(jnp.float32)
        for b in range(1, _NBINS):
            nonempty += (cnts[b] > 0.0).astype(jnp.float32)
        total = sums[0] * 0.0
        for b in range(_NBINS):
            gd = jnp.maximum(cnts[b] * nonempty, 1.0)
            total += sums[b] * (n_total / gd)
        out_ref[0, 0] = -total / n_total


def kernel(x, target):
    out = pl.pallas_call(
        _ghme_body,
        grid=(_ROWS // _BR,),
        in_specs=[
            pl.BlockSpec((_BR, _COLS), lambda i: (i, 0)),
            pl.BlockSpec((_BR, _COLS), lambda i: (i, 0)),
        ],
        out_specs=pl.BlockSpec((1, 1), lambda i: (0, 0), memory_space=pltpu.SMEM),
        out_shape=jax.ShapeDtypeStruct((1, 1), jnp.float32),
        scratch_shapes=[
            pltpu.VMEM((_NACC, 8, 128), jnp.float32),
        ],
    )(x, target)
    return out.reshape(())


# speedup vs baseline: 18.1652x; 1.0184x over previous
import jax
import jax.numpy as jnp
from jax.experimental import pallas as pl
from jax.experimental.pallas import tpu as pltpu

_NBINS = 10
_ROWS = 16384
_COLS = 1000
_BR = 128
_NACC = 2 * (_NBINS - 1) + 1
_NCH = (_COLS + 127) // 128


def _ghme_body(x_ref, t_ref, out_ref, acc_ref):
    i = pl.program_id(0)
    nsteps = pl.num_programs(0)

    @pl.when(i == 0)
    def _init():
        acc_ref[...] = jnp.zeros_like(acc_ref)

    zero = jnp.zeros((8, 128), jnp.float32)

    tail = _COLS - (_NCH - 1) * 128
    pad = jnp.zeros((8, 128 - tail), jnp.float32)

    def row_group(rg, carry):
        accs = list(carry)
        r0 = rg * 8
        for c in range(_NCH):
            c0 = c * 128
            w = 128 if c < _NCH - 1 else tail
            x = x_ref[pl.ds(r0, 8), c0:c0 + w]
            t = t_ref[pl.ds(r0, 8), c0:c0 + w]
            g = jnp.abs(x - t)
            idx = jnp.floor(g * (_NBINS - 0.0001))
            lx = jnp.log(x)
            l1x = jnp.log(1.0 - x)
            nloss = t * (lx - l1x) + l1x
            if w < 128:
                idx = jnp.concatenate([idx, pad], axis=1)
                nloss = jnp.concatenate([nloss, pad], axis=1)
            accs[_NACC - 1] = accs[_NACC - 1] + nloss
            for j in range(_NBINS - 1):
                m01 = jnp.where(idx == jnp.float32(j + 1), 1.0, 0.0)
                accs[j] = accs[j] + m01 * nloss
                accs[_NBINS - 1 + j] = accs[_NBINS - 1 + j] + m01
        return tuple(accs)

    accs = jax.lax.fori_loop(0, _BR // 8, row_group, (zero,) * _NACC,
                             unroll=2)
    for k in range(_NACC):
        acc_ref[k] += accs[k]

    @pl.when(i == nsteps - 1)
    def _fini():
        n_total = jnp.float32(_ROWS * _COLS)
        sums = [jnp.sum(acc_ref[j]) for j in range(_NBINS - 1)]
        cnts = [jnp.sum(acc_ref[_NBINS - 1 + j]) for j in range(_NBINS - 1)]
        tot_s = jnp.sum(acc_ref[_NACC - 1])
        c_rest = cnts[0]
        s_rest = sums[0]
        for j in range(1, _NBINS - 1):
            c_rest += cnts[j]
            s_rest += sums[j]
        cnts = [n_total - c_rest] + cnts
        sums = [tot_s - s_rest] + sums
        nonempty = (cnts[0] > 0.0).astype(jnp.float32)
        for b in range(1, _NBINS):
            nonempty += (cnts[b] > 0.0).astype(jnp.float32)
        total = sums[0] * 0.0
        for b in range(_NBINS):
            gd = jnp.maximum(cnts[b] * nonempty, 1.0)
            total += sums[b] * (n_total / gd)
        out_ref[0, 0] = -total / n_total


def kernel(x, target):
    out = pl.pallas_call(
        _ghme_body,
        grid=(_ROWS // _BR,),
        in_specs=[
            pl.BlockSpec((_BR, _COLS), lambda i: (i, 0)),
            pl.BlockSpec((_BR, _COLS), lambda i: (i, 0)),
        ],
        out_specs=pl.BlockSpec((1, 1), lambda i: (0, 0), memory_space=pltpu.SMEM),
        out_shape=jax.ShapeDtypeStruct((1, 1), jnp.float32),
        scratch_shapes=[
            pltpu.VMEM((_NACC, 8, 128), jnp.float32),
        ],
    )(x, target)
    return out.reshape(())


# speedup vs baseline: 36.0271x; 1.9833x over previous
import jax
import jax.numpy as jnp
from jax.experimental import pallas as pl
from jax.experimental.pallas import tpu as pltpu

_NBINS = 10
_ROWS = 1000
_COLS = 16384
_BC = 1024
_NACC = 2 * (_NBINS - 1) + 1


def _ghme_body(x_ref, t_ref, out_ref, acc_ref):
    i = pl.program_id(0)
    nsteps = pl.num_programs(0)

    @pl.when(i == 0)
    def _init():
        acc_ref[...] = jnp.zeros_like(acc_ref)

    zero = jnp.zeros((8, 128), jnp.float32)

    def row_group(rg, carry):
        accs = list(carry)
        r0 = pl.multiple_of(rg * 8, 8)
        for c in range(_BC // 128):
            x = x_ref[pl.ds(r0, 8), c * 128:(c + 1) * 128]
            t = t_ref[pl.ds(r0, 8), c * 128:(c + 1) * 128]
            g = jnp.abs(x - t)
            idx = jnp.floor(g * (_NBINS - 0.0001))
            lx = jnp.log(x)
            l1x = jnp.log(1.0 - x)
            nloss = t * (lx - l1x) + l1x
            accs[_NACC - 1] = accs[_NACC - 1] + nloss
            for j in range(_NBINS - 1):
                m01 = jnp.where(idx == jnp.float32(j + 1), 1.0, 0.0)
                accs[j] = accs[j] + m01 * nloss
                accs[_NBINS - 1 + j] = accs[_NBINS - 1 + j] + m01
        return tuple(accs)

    accs = jax.lax.fori_loop(0, _ROWS // 8, row_group, (zero,) * _NACC,
                             unroll=2)
    for k in range(_NACC):
        acc_ref[k] += accs[k]

    @pl.when(i == nsteps - 1)
    def _fini():
        n_total = jnp.float32(_ROWS * _COLS)
        sums = [jnp.sum(acc_ref[j]) for j in range(_NBINS - 1)]
        cnts = [jnp.sum(acc_ref[_NBINS - 1 + j]) for j in range(_NBINS - 1)]
        tot_s = jnp.sum(acc_ref[_NACC - 1])
        c_rest = cnts[0]
        s_rest = sums[0]
        for j in range(1, _NBINS - 1):
            c_rest += cnts[j]
            s_rest += sums[j]
        cnts = [n_total - c_rest] + cnts
        sums = [tot_s - s_rest] + sums
        nonempty = (cnts[0] > 0.0).astype(jnp.float32)
        for b in range(1, _NBINS):
            nonempty += (cnts[b] > 0.0).astype(jnp.float32)
        total = sums[0] * 0.0
        for b in range(_NBINS):
            gd = jnp.maximum(cnts[b] * nonempty, 1.0)
            total += sums[b] * (n_total / gd)
        out_ref[0, 0] = -total / n_total


def kernel(x, target):
    xt = jnp.swapaxes(x, 0, 1)
    tt = jnp.swapaxes(target, 0, 1)
    out = pl.pallas_call(
        _ghme_body,
        grid=(_COLS // _BC,),
        in_specs=[
            pl.BlockSpec((_ROWS, _BC), lambda i: (0, i)),
            pl.BlockSpec((_ROWS, _BC), lambda i: (0, i)),
        ],
        out_specs=pl.BlockSpec((1, 1), lambda i: (0, 0), memory_space=pltpu.SMEM),
        out_shape=jax.ShapeDtypeStruct((1, 1), jnp.float32),
        scratch_shapes=[
            pltpu.VMEM((_NACC, 8, 128), jnp.float32),
        ],
    )(xt, tt)
    return out.reshape(())


# speedup vs baseline: 37.4417x; 1.0393x over previous
import jax
import jax.numpy as jnp
from jax.experimental import pallas as pl
from jax.experimental.pallas import tpu as pltpu

_NBINS = 10
_ROWS = 1000
_COLS = 16384
_BC = 2048
_NACC = 2 * (_NBINS - 1) + 1


def _ghme_body(x_ref, t_ref, out_ref, acc_ref):
    i = pl.program_id(0)
    nsteps = pl.num_programs(0)

    @pl.when(i == 0)
    def _init():
        acc_ref[...] = jnp.zeros_like(acc_ref)

    zero = jnp.zeros((8, 128), jnp.float32)

    def row_group(rg, carry):
        accs = list(carry)
        r0 = pl.multiple_of(rg * 8, 8)
        for c in range(_BC // 128):
            x = x_ref[pl.ds(r0, 8), c * 128:(c + 1) * 128]
            t = t_ref[pl.ds(r0, 8), c * 128:(c + 1) * 128]
            g = jnp.abs(x - t)
            idx = jnp.floor(g * (_NBINS - 0.0001))
            lx = jnp.log(x)
            l1x = jnp.log(1.0 - x)
            nloss = t * (lx - l1x) + l1x
            accs[_NACC - 1] = accs[_NACC - 1] + nloss
            for j in range(_NBINS - 1):
                m01 = jnp.where(idx == jnp.float32(j + 1), 1.0, 0.0)
                accs[j] = accs[j] + m01 * nloss
                accs[_NBINS - 1 + j] = accs[_NBINS - 1 + j] + m01
        return tuple(accs)

    accs = jax.lax.fori_loop(0, _ROWS // 8, row_group, (zero,) * _NACC,
                             unroll=25)
    for k in range(_NACC):
        acc_ref[k] += accs[k]

    @pl.when(i == nsteps - 1)
    def _fini():
        n_total = jnp.float32(_ROWS * _COLS)
        sums = [jnp.sum(acc_ref[j]) for j in range(_NBINS - 1)]
        cnts = [jnp.sum(acc_ref[_NBINS - 1 + j]) for j in range(_NBINS - 1)]
        tot_s = jnp.sum(acc_ref[_NACC - 1])
        c_rest = cnts[0]
        s_rest = sums[0]
        for j in range(1, _NBINS - 1):
            c_rest += cnts[j]
            s_rest += sums[j]
        cnts = [n_total - c_rest] + cnts
        sums = [tot_s - s_rest] + sums
        nonempty = (cnts[0] > 0.0).astype(jnp.float32)
        for b in range(1, _NBINS):
            nonempty += (cnts[b] > 0.0).astype(jnp.float32)
        total = sums[0] * 0.0
        for b in range(_NBINS):
            gd = jnp.maximum(cnts[b] * nonempty, 1.0)
            total += sums[b] * (n_total / gd)
        out_ref[0, 0] = -total / n_total


def kernel(x, target):
    xt = jnp.swapaxes(x, 0, 1)
    tt = jnp.swapaxes(target, 0, 1)
    out = pl.pallas_call(
        _ghme_body,
        grid=(_COLS // _BC,),
        in_specs=[
            pl.BlockSpec((_ROWS, _BC), lambda i: (0, i)),
            pl.BlockSpec((_ROWS, _BC), lambda i: (0, i)),
        ],
        out_specs=pl.BlockSpec((1, 1), lambda i: (0, 0), memory_space=pltpu.SMEM),
        out_shape=jax.ShapeDtypeStruct((1, 1), jnp.float32),
        scratch_shapes=[
            pltpu.VMEM((_NACC, 8, 128), jnp.float32),
        ],
    )(xt, tt)
    return out.reshape(())


# speedup vs baseline: 37.9442x; 1.0134x over previous
import jax
import jax.numpy as jnp
from jax.experimental import pallas as pl
from jax.experimental.pallas import tpu as pltpu

_NBINS = 10
_ROWS = 1000
_COLS = 16384
_BC = 1024
_NACC = 2 * (_NBINS - 1) + 1


def _ghme_body(x_ref, t_ref, out_ref, acc_ref):
    i = pl.program_id(0)
    nsteps = pl.num_programs(0)

    @pl.when(i == 0)
    def _init():
        acc_ref[...] = jnp.zeros_like(acc_ref)

    zero = jnp.zeros((8, 128), jnp.float32)

    def row_group(rg, carry):
        accs = list(carry)
        r0 = pl.multiple_of(rg * 8, 8)
        for c in range(_BC // 128):
            x = x_ref[pl.ds(r0, 8), c * 128:(c + 1) * 128]
            t = t_ref[pl.ds(r0, 8), c * 128:(c + 1) * 128]
            g = jnp.abs(x - t)
            idx = jnp.floor(g * (_NBINS - 0.0001))
            lx = jnp.log(x)
            l1x = jnp.log(1.0 - x)
            nloss = t * (lx - l1x) + l1x
            accs[_NACC - 1] = accs[_NACC - 1] + nloss
            for j in range(_NBINS - 1):
                m01 = jnp.where(idx == jnp.float32(j + 1), 1.0, 0.0)
                accs[j] = accs[j] + m01 * nloss
                accs[_NBINS - 1 + j] = accs[_NBINS - 1 + j] + m01
        return tuple(accs)

    accs = jax.lax.fori_loop(0, _ROWS // 8, row_group, (zero,) * _NACC,
                             unroll=25)
    for k in range(_NACC):
        acc_ref[k] += accs[k]

    @pl.when(i == nsteps - 1)
    def _fini():
        n_total = jnp.float32(_ROWS * _COLS)
        sums = [jnp.sum(acc_ref[j]) for j in range(_NBINS - 1)]
        cnts = [jnp.sum(acc_ref[_NBINS - 1 + j]) for j in range(_NBINS - 1)]
        tot_s = jnp.sum(acc_ref[_NACC - 1])
        c_rest = cnts[0]
        s_rest = sums[0]
        for j in range(1, _NBINS - 1):
            c_rest += cnts[j]
            s_rest += sums[j]
        cnts = [n_total - c_rest] + cnts
        sums = [tot_s - s_rest] + sums
        nonempty = (cnts[0] > 0.0).astype(jnp.float32)
        for b in range(1, _NBINS):
            nonempty += (cnts[b] > 0.0).astype(jnp.float32)
        total = sums[0] * 0.0
        for b in range(_NBINS):
            gd = jnp.maximum(cnts[b] * nonempty, 1.0)
            total += sums[b] * (n_total / gd)
        out_ref[0, 0] = -total / n_total


def kernel(x, target):
    xt = jnp.swapaxes(x, 0, 1)
    tt = jnp.swapaxes(target, 0, 1)
    out = pl.pallas_call(
        _ghme_body,
        grid=(_COLS // _BC,),
        in_specs=[
            pl.BlockSpec((_ROWS, _BC), lambda i: (0, i)),
            pl.BlockSpec((_ROWS, _BC), lambda i: (0, i)),
        ],
        out_specs=pl.BlockSpec((1, 1), lambda i: (0, 0), memory_space=pltpu.SMEM),
        out_shape=jax.ShapeDtypeStruct((1, 1), jnp.float32),
        scratch_shapes=[
            pltpu.VMEM((_NACC, 8, 128), jnp.float32),
        ],
    )(xt, tt)
    return out.reshape(())
